# baseline (speedup 1.0000x reference)
import functools

import jax
import jax.numpy as jnp
from jax import lax
from jax.experimental import pallas as pl
from jax.experimental.pallas import tpu as pltpu
from jax.experimental.pallas import tpu_sc as plsc

_COMMITMENT_COST = 0.25


def _encode_vq_body(x_ref, eps_ref,
                    mw1, mb1, mw2, mb2, mw3, mb3,
                    lw1, lb1, lw2, lb2, lw3, lb3,
                    cb_ref,
                    z_ref, idx_ref, kl_ref,
                    cn_s, m_s, i_s):
    ct = pl.program_id(1)
    ib = pl.program_id(0)
    n_ct = pl.num_programs(1)

    @pl.when(ct == 0)
    def _():
        x = x_ref[...]
        h = jnp.tanh(jnp.dot(x, mw1[...], preferred_element_type=jnp.float32) + mb1[...])
        h = jnp.tanh(jnp.dot(h, mw2[...], preferred_element_type=jnp.float32) + mb2[...])
        mu = jnp.dot(h, mw3[...], preferred_element_type=jnp.float32) + mb3[...]

        g = jnp.tanh(jnp.dot(x, lw1[...], preferred_element_type=jnp.float32) + lb1[...])
        g = jnp.tanh(jnp.dot(g, lw2[...], preferred_element_type=jnp.float32) + lb2[...])
        logvar = jnp.dot(g, lw3[...], preferred_element_type=jnp.float32) + lb3[...]

        z_ref[...] = -2.0 * (mu + jnp.exp(0.5 * logvar) * eps_ref[...])

        kl_part = jnp.sum(1.0 + logvar - mu * mu - jnp.exp(logvar), keepdims=True)

        @pl.when(ib == 0)
        def _():
            kl_ref[...] = jnp.zeros((1, 1), jnp.float32)

        kl_ref[...] += kl_part
        m_s[...] = jnp.full(m_s.shape, jnp.inf, jnp.float32)
        i_s[...] = jnp.zeros(i_s.shape, jnp.int32)

    c = cb_ref[...]
    cb_tile = c.shape[1]

    @pl.when(ib == 0)
    def _():
        cn_s[:, pl.ds(ct * cb_tile, cb_tile)] = jnp.sum(
            c * c, axis=0, keepdims=True)

    cn = cn_s[:, pl.ds(ct * cb_tile, cb_tile)]
    s = jnp.dot(z_ref[...], c, preferred_element_type=jnp.float32)
    d = s + cn
    b_blk = d.shape[0]
    m = m_s[...]
    idx = i_s[...]
    lane = lax.broadcasted_iota(jnp.int32, (b_blk, 128), 1)
    for blk in range(cb_tile // 128):
        db = d[:, blk * 128:(blk + 1) * 128]
        cand = lane + (ct * cb_tile + blk * 128)
        better = db < m
        m = jnp.where(better, db, m)
        idx = jnp.where(better, cand, idx)
    m_s[...] = m
    i_s[...] = idx

    @pl.when(ct == n_ct - 1)
    def _():
        dmin = jnp.min(m, axis=1, keepdims=True)
        best = jnp.min(jnp.where(m == dmin, idx, jnp.int32(2**30)), axis=1)
        idx_ref[...] = best.reshape(idx_ref.shape)


def _decode_body(q_ref, z2_ref, dw1, db1, dw2, db2, dw3, db3,
                 xr_ref, vq_ref):
    q = q_ref[...]
    diff = q + 0.5 * z2_ref[...]
    vq_part = jnp.sum(diff * diff, keepdims=True)

    @pl.when(pl.program_id(0) == 0)
    def _():
        vq_ref[...] = jnp.zeros((1, 1), jnp.float32)

    vq_ref[...] += vq_part

    h = jnp.tanh(jnp.dot(q, dw1[...], preferred_element_type=jnp.float32) + db1[...])
    h = jnp.tanh(jnp.dot(h, dw2[...], preferred_element_type=jnp.float32) + db2[...])
    xr_ref[...] = jnp.dot(h, dw3[...], preferred_element_type=jnp.float32) + db3[...]


def _make_sc_gather(num_emb, emb_dim, batch):
    info = plsc.get_sparse_core_info()
    nc, ns = info.num_cores, info.num_subcores
    nw = nc * ns
    assert batch % nw == 0
    b_per_w = batch // nw
    mesh = plsc.VectorSubcoreMesh(core_axis_name="c", subcore_axis_name="s")

    @functools.partial(
        pl.kernel, mesh=mesh,
        out_type=jax.ShapeDtypeStruct((batch, emb_dim), jnp.float32),
        scratch_types=[
            pltpu.VMEM((b_per_w,), jnp.int32),
            pltpu.VMEM((b_per_w, emb_dim), jnp.float32),
            pltpu.SemaphoreType.DMA,
        ],
    )
    def gather(table_hbm, idx_hbm, out_hbm, idx_v, rows_v, sem):
        wid = lax.axis_index("s") * nc + lax.axis_index("c")
        base = wid * b_per_w
        pltpu.sync_copy(idx_hbm.at[pl.ds(base, b_per_w)], idx_v)
        pltpu.async_copy(table_hbm.at[idx_v], rows_v, sem).wait()
        pltpu.sync_copy(rows_v, out_hbm.at[pl.ds(base, b_per_w)])

    return gather


def kernel(x, eps, mu_params, logvar_params, dec_params, codebook):
    batch, in_dim = x.shape
    emb_dim = eps.shape[1]
    num_emb = codebook.shape[0]

    b_blk = 4096
    nb = batch // b_blk
    cb_tile = 512
    n_ct = num_emb // cb_tile

    def row(p):
        out = []
        for w, b in p:
            out.append(w)
            out.append(b.reshape(1, -1))
        return out

    mu_flat = row(mu_params)
    lv_flat = row(logvar_params)
    dec_flat = row(dec_params)

    def wspec2(a):
        return pl.BlockSpec(a.shape, lambda i, j: (0,) * a.ndim)

    z, idx3, kl_sum = pl.pallas_call(
        _encode_vq_body,
        grid=(nb, n_ct),
        in_specs=[
            pl.BlockSpec((b_blk, in_dim), lambda i, j: (i, 0)),
            pl.BlockSpec((b_blk, emb_dim), lambda i, j: (i, 0)),
            *[wspec2(a) for a in mu_flat],
            *[wspec2(a) for a in lv_flat],
            pl.BlockSpec((emb_dim, cb_tile), lambda i, j: (0, j)),
        ],
        out_specs=[
            pl.BlockSpec((b_blk, emb_dim), lambda i, j: (i, 0)),
            pl.BlockSpec((1, 1, b_blk), lambda i, j: (i, 0, 0)),
            pl.BlockSpec((1, 1), lambda i, j: (0, 0)),
        ],
        out_shape=[
            jax.ShapeDtypeStruct((batch, emb_dim), jnp.float32),
            jax.ShapeDtypeStruct((nb, 1, b_blk), jnp.int32),
            jax.ShapeDtypeStruct((1, 1), jnp.float32),
        ],
        scratch_shapes=[
            pltpu.VMEM((1, num_emb), jnp.float32),
            pltpu.VMEM((b_blk, 128), jnp.float32),
            pltpu.VMEM((b_blk, 128), jnp.int32),
        ],
    )(x, eps, *mu_flat, *lv_flat, codebook.T)

    idx = idx3.reshape(batch)
    quantized = _make_sc_gather(num_emb, emb_dim, batch)(codebook, idx)

    def wspec1(a):
        return pl.BlockSpec(a.shape, lambda i: (0,) * a.ndim)

    xr, vq_sum = pl.pallas_call(
        _decode_body,
        grid=(nb,),
        in_specs=[
            pl.BlockSpec((b_blk, emb_dim), lambda i: (i, 0)),
            pl.BlockSpec((b_blk, emb_dim), lambda i: (i, 0)),
            *[wspec1(a) for a in dec_flat],
        ],
        out_specs=[
            pl.BlockSpec((b_blk, in_dim), lambda i: (i, 0)),
            pl.BlockSpec((1, 1), lambda i: (0, 0)),
        ],
        out_shape=[
            jax.ShapeDtypeStruct((batch, in_dim), jnp.float32),
            jax.ShapeDtypeStruct((1, 1), jnp.float32),
        ],
    )(quantized, z, *dec_flat)

    vq_loss = (1.0 + _COMMITMENT_COST) * vq_sum[0, 0] / (batch * emb_dim)
    kl_loss = -0.5 * kl_sum[0, 0]
    return xr, vq_loss, kl_loss


# speedup vs baseline: 1.0148x; 1.0148x over previous
import functools

import jax
import jax.numpy as jnp
from jax import lax
from jax.experimental import pallas as pl
from jax.experimental.pallas import tpu as pltpu
from jax.experimental.pallas import tpu_sc as plsc

_COMMITMENT_COST = 0.25


def _encode_vq_body(x_ref, eps_ref,
                    mw1, mb1, mw2, mb2, mw3, mb3,
                    lw1, lb1, lw2, lb2, lw3, lb3,
                    cb_ref,
                    z_ref, idx_ref, kl_ref,
                    cn_s, m_s, i_s):
    ct = pl.program_id(1)
    ib = pl.program_id(0)
    n_ct = pl.num_programs(1)

    @pl.when(ct == 0)
    def _():
        x = x_ref[...]
        h = jnp.tanh(jnp.dot(x, mw1[...], preferred_element_type=jnp.float32) + mb1[...])
        h = jnp.tanh(jnp.dot(h, mw2[...], preferred_element_type=jnp.float32) + mb2[...])
        mu = jnp.dot(h, mw3[...], preferred_element_type=jnp.float32) + mb3[...]

        g = jnp.tanh(jnp.dot(x, lw1[...], preferred_element_type=jnp.float32) + lb1[...])
        g = jnp.tanh(jnp.dot(g, lw2[...], preferred_element_type=jnp.float32) + lb2[...])
        logvar = jnp.dot(g, lw3[...], preferred_element_type=jnp.float32) + lb3[...]

        z_ref[...] = -2.0 * (mu + jnp.exp(0.5 * logvar) * eps_ref[...])

        kl_part = jnp.sum(1.0 + logvar - mu * mu - jnp.exp(logvar), keepdims=True)

        @pl.when(ib == 0)
        def _():
            kl_ref[...] = jnp.zeros((1, 1), jnp.float32)

        kl_ref[...] += kl_part
        m_s[...] = jnp.full(m_s.shape, jnp.inf, jnp.float32)
        i_s[...] = jnp.zeros(i_s.shape, jnp.int32)

    c = cb_ref[...]
    cb_tile = c.shape[1]

    @pl.when(ib == 0)
    def _():
        cn_s[:, pl.ds(ct * cb_tile, cb_tile)] = jnp.sum(
            c * c, axis=0, keepdims=True)

    cn = cn_s[:, pl.ds(ct * cb_tile, cb_tile)]
    s = jnp.dot(z_ref[...], c, preferred_element_type=jnp.float32)
    d = s + cn
    b_blk = d.shape[0]
    m = m_s[...]
    idx = i_s[...]
    lane = lax.broadcasted_iota(jnp.int32, (b_blk, 128), 1)
    for blk in range(cb_tile // 128):
        db = d[:, blk * 128:(blk + 1) * 128]
        cand = lane + (ct * cb_tile + blk * 128)
        better = db < m
        m = jnp.where(better, db, m)
        idx = jnp.where(better, cand, idx)
    m_s[...] = m
    i_s[...] = idx

    @pl.when(ct == n_ct - 1)
    def _():
        dmin = jnp.min(m, axis=1, keepdims=True)
        best = jnp.min(jnp.where(m == dmin, idx, jnp.int32(2**30)), axis=1)
        idx_ref[...] = best.reshape(idx_ref.shape)


def _decode_body(q_ref, z2_ref, dw1, db1, dw2, db2, dw3, db3,
                 xr_ref, vq_ref):
    q = q_ref[...]
    diff = q + 0.5 * z2_ref[...]
    vq_part = jnp.sum(diff * diff, keepdims=True)

    @pl.when(pl.program_id(0) == 0)
    def _():
        vq_ref[...] = jnp.zeros((1, 1), jnp.float32)

    vq_ref[...] += vq_part

    h = jnp.tanh(jnp.dot(q, dw1[...], preferred_element_type=jnp.float32) + db1[...])
    h = jnp.tanh(jnp.dot(h, dw2[...], preferred_element_type=jnp.float32) + db2[...])
    xr_ref[...] = jnp.dot(h, dw3[...], preferred_element_type=jnp.float32) + db3[...]


def _make_sc_gather(num_emb, emb_dim, batch):
    info = plsc.get_sparse_core_info()
    nc, ns = info.num_cores, info.num_subcores
    nw = nc * ns
    assert batch % nw == 0
    b_per_w = batch // nw
    mesh = plsc.VectorSubcoreMesh(core_axis_name="c", subcore_axis_name="s")

    @functools.partial(
        pl.kernel, mesh=mesh,
        out_type=jax.ShapeDtypeStruct((batch, emb_dim), jnp.float32),
        scratch_types=[
            pltpu.VMEM((b_per_w,), jnp.int32),
            pltpu.VMEM((b_per_w, emb_dim), jnp.float32),
            pltpu.SemaphoreType.DMA,
        ],
    )
    def gather(table_hbm, idx_hbm, out_hbm, idx_v, rows_v, sem):
        wid = lax.axis_index("s") * nc + lax.axis_index("c")
        base = wid * b_per_w
        pltpu.sync_copy(idx_hbm.at[pl.ds(base, b_per_w)], idx_v)
        pltpu.async_copy(table_hbm.at[idx_v], rows_v, sem).wait()
        pltpu.sync_copy(rows_v, out_hbm.at[pl.ds(base, b_per_w)])

    return gather


def kernel(x, eps, mu_params, logvar_params, dec_params, codebook):
    batch, in_dim = x.shape
    emb_dim = eps.shape[1]
    num_emb = codebook.shape[0]

    b_blk = 2048
    nb = batch // b_blk
    cb_tile = 512
    n_ct = num_emb // cb_tile

    def row(p):
        out = []
        for w, b in p:
            out.append(w)
            out.append(b.reshape(1, -1))
        return out

    mu_flat = row(mu_params)
    lv_flat = row(logvar_params)
    dec_flat = row(dec_params)

    def wspec2(a):
        return pl.BlockSpec(a.shape, lambda i, j: (0,) * a.ndim)

    z, idx3, kl_sum = pl.pallas_call(
        _encode_vq_body,
        grid=(nb, n_ct),
        in_specs=[
            pl.BlockSpec((b_blk, in_dim), lambda i, j: (i, 0)),
            pl.BlockSpec((b_blk, emb_dim), lambda i, j: (i, 0)),
            *[wspec2(a) for a in mu_flat],
            *[wspec2(a) for a in lv_flat],
            pl.BlockSpec((emb_dim, cb_tile), lambda i, j: (0, j)),
        ],
        out_specs=[
            pl.BlockSpec((b_blk, emb_dim), lambda i, j: (i, 0)),
            pl.BlockSpec((1, 1, b_blk), lambda i, j: (i, 0, 0)),
            pl.BlockSpec((1, 1), lambda i, j: (0, 0)),
        ],
        out_shape=[
            jax.ShapeDtypeStruct((batch, emb_dim), jnp.float32),
            jax.ShapeDtypeStruct((nb, 1, b_blk), jnp.int32),
            jax.ShapeDtypeStruct((1, 1), jnp.float32),
        ],
        scratch_shapes=[
            pltpu.VMEM((1, num_emb), jnp.float32),
            pltpu.VMEM((b_blk, 128), jnp.float32),
            pltpu.VMEM((b_blk, 128), jnp.int32),
        ],
    )(x, eps, *mu_flat, *lv_flat, codebook.T)

    idx = idx3.reshape(batch)
    quantized = _make_sc_gather(num_emb, emb_dim, batch)(codebook, idx)

    def wspec1(a):
        return pl.BlockSpec(a.shape, lambda i: (0,) * a.ndim)

    xr, vq_sum = pl.pallas_call(
        _decode_body,
        grid=(nb,),
        in_specs=[
            pl.BlockSpec((b_blk, emb_dim), lambda i: (i, 0)),
            pl.BlockSpec((b_blk, emb_dim), lambda i: (i, 0)),
            *[wspec1(a) for a in dec_flat],
        ],
        out_specs=[
            pl.BlockSpec((b_blk, in_dim), lambda i: (i, 0)),
            pl.BlockSpec((1, 1), lambda i: (0, 0)),
        ],
        out_shape=[
            jax.ShapeDtypeStruct((batch, in_dim), jnp.float32),
            jax.ShapeDtypeStruct((1, 1), jnp.float32),
        ],
    )(quantized, z, *dec_flat)

    vq_loss = (1.0 + _COMMITMENT_COST) * vq_sum[0, 0] / (batch * emb_dim)
    kl_loss = -0.5 * kl_sum[0, 0]
    return xr, vq_loss, kl_loss
